# baseline (speedup 1.0000x reference)
import functools

import jax
import jax.numpy as jnp
from jax import lax
from jax.experimental import pallas as pl
from jax.experimental.pallas import tpu as pltpu
from jax.experimental.pallas import tpu_sc as plsc

P = 512
D = 256
N = 1024
COMMIT_W = 0.1
ENT_LAMBDA = 200000.0
BR = 8
NBLK = N // BR

_F32 = jnp.float32


def _seq_reduce_t(prodT):
    acc = prodT[0:8, :]
    for k in range(1, D // 8):
        acc = acc + prodT[8 * k:8 * k + 8, :]
    s = acc[0:4, :] + acc[4:8, :]
    s = s[0:2, :] + s[2:4, :]
    return s[0:1, :] + s[1:2, :]


def _tc_body(x_ref, ct_ref, a_ref, idx_ref, commit_ref, cb_ref,
             cnt_scr, acc_scr):
    i = pl.program_id(0)
    ccT = ct_ref[...]
    aa = a_ref[...]

    @pl.when(i == 0)
    def _prologue():
        cnt_scr[...] = jnp.zeros((1, P), _F32)
        acc_scr[...] = jnp.zeros((1, 1), _F32)

    rows = []
    for r in range(BR):
        col = x_ref[r:r + 1, :]
        diffT = jnp.broadcast_to(col.reshape(D, 1), (D, P)) - ccT
        leftT = lax.dot_general(aa, diffT, (((0,), (0,)), ((), ())),
                                preferred_element_type=_F32)
        rows.append(_seq_reduce_t(leftT * diffT))
    scores = jnp.concatenate(rows, axis=0)

    smin = jnp.min(scores, axis=1, keepdims=True)
    pio = lax.broadcasted_iota(jnp.int32, (BR, P), 1)
    idx = jnp.min(jnp.where(scores == smin, pio, P), axis=1, keepdims=True)
    idx_ref[...] = idx

    acc_scr[...] += jnp.full((1, 1), jnp.sum(smin), _F32)
    onehot = (idx == pio).astype(_F32)
    cnt_scr[...] += jnp.sum(onehot, axis=0, keepdims=True)

    @pl.when(i == NBLK - 1)
    def _epilogue():
        M = acc_scr[0, 0] * (1.0 / N)
        counts = cnt_scr[...]
        probs = counts / (jnp.float32(N) + 1e-8)
        entropy = -jnp.sum(probs * jnp.log(probs + 1e-8))
        commit_ref[...] = jnp.full((1, 1), COMMIT_W * M, _F32)
        cb_ref[...] = jnp.full((1, 1), M - ENT_LAMBDA * entropy, _F32)


_tc_call = pl.pallas_call(
    _tc_body,
    grid=(NBLK,),
    in_specs=[
        pl.BlockSpec((BR, D), lambda i: (i, 0)),
        pl.BlockSpec((D, P), lambda i: (0, 0)),
        pl.BlockSpec((D, D), lambda i: (0, 0)),
    ],
    out_specs=(
        pl.BlockSpec((BR, 1), lambda i: (i, 0)),
        pl.BlockSpec((1, 1), lambda i: (0, 0)),
        pl.BlockSpec((1, 1), lambda i: (0, 0)),
    ),
    out_shape=(
        jax.ShapeDtypeStruct((N, 1), jnp.int32),
        jax.ShapeDtypeStruct((1, 1), _F32),
        jax.ShapeDtypeStruct((1, 1), _F32),
    ),
    scratch_shapes=[
        pltpu.VMEM((1, P), _F32),
        pltpu.VMEM((1, 1), _F32),
    ],
)


def _sc_gather(codebook, idx):
    info = plsc.get_sparse_core_info()
    nw = info.num_cores * info.num_subcores
    b_per_w = N // nw
    mesh = plsc.VectorSubcoreMesh(core_axis_name="c", subcore_axis_name="s")

    @functools.partial(
        pl.kernel, mesh=mesh,
        out_type=jax.ShapeDtypeStruct((N, D), _F32),
        scratch_types=[
            pltpu.VMEM((b_per_w,), jnp.int32),
            pltpu.VMEM((b_per_w, D), _F32),
            pltpu.SemaphoreType.DMA,
        ],
    )
    def k(table_hbm, idx_hbm, out_hbm, idx_v, rows_v, sem):
        wid = lax.axis_index("s") * info.num_cores + lax.axis_index("c")
        base = wid * b_per_w
        pltpu.sync_copy(idx_hbm.at[pl.ds(base, b_per_w)], idx_v)
        pltpu.async_copy(table_hbm.at[idx_v], rows_v, sem).wait()
        pltpu.sync_copy(rows_v, out_hbm.at[pl.ds(base, b_per_w)])

    return k(codebook, idx)


def kernel(input_data, codebook):
    mu = jnp.mean(codebook, axis=0, keepdims=True)
    cen = codebook - mu
    cov = cen.T @ cen / (P - 1)
    cov = cov + 0.001 * jnp.eye(D, dtype=_F32)
    inv_cov = jax.lax.stop_gradient(jnp.linalg.inv(cov))

    idx2d, commit, cb = _tc_call(input_data, codebook.T, inv_cov)
    idx = idx2d.reshape(N)
    quantized = _sc_gather(codebook, idx)
    return (quantized, commit[0, 0], cb[0, 0], idx)


# speedup vs baseline: 1.1606x; 1.1606x over previous
import functools

import jax
import jax.numpy as jnp
from jax import lax
from jax.experimental import pallas as pl
from jax.experimental.pallas import tpu as pltpu
from jax.experimental.pallas import tpu_sc as plsc

P = 512
D = 256
N = 1024
COMMIT_W = 0.1
ENT_LAMBDA = 200000.0
BR = 32
NBLK = N // BR

_F32 = jnp.float32


def _seq_reduce_t(prodT):
    acc = prodT[0:8, :]
    for k in range(1, D // 8):
        acc = acc + prodT[8 * k:8 * k + 8, :]
    s = acc[0:4, :] + acc[4:8, :]
    s = s[0:2, :] + s[2:4, :]
    return s[0:1, :] + s[1:2, :]


def _tc_body(x_ref, ct_ref, a_ref, idx_ref, commit_ref, cb_ref,
             cnt_scr, acc_scr):
    i = pl.program_id(0)
    ccT = ct_ref[...]
    aa = a_ref[...]

    @pl.when(i == 0)
    def _prologue():
        cnt_scr[...] = jnp.zeros((1, P), _F32)
        acc_scr[...] = jnp.zeros((1, 1), _F32)

    rows = []
    for r in range(BR):
        col = x_ref[r:r + 1, :]
        diffT = jnp.broadcast_to(col.reshape(D, 1), (D, P)) - ccT
        leftT = lax.dot_general(aa, diffT, (((0,), (0,)), ((), ())),
                                preferred_element_type=_F32)
        rows.append(_seq_reduce_t(leftT * diffT))
    scores = jnp.concatenate(rows, axis=0)

    smin = jnp.min(scores, axis=1, keepdims=True)
    pio = lax.broadcasted_iota(jnp.int32, (BR, P), 1)
    idx = jnp.min(jnp.where(scores == smin, pio, P), axis=1, keepdims=True)
    idx_ref[...] = idx

    acc_scr[...] += jnp.full((1, 1), jnp.sum(smin), _F32)
    onehot = (idx == pio).astype(_F32)
    cnt_scr[...] += jnp.sum(onehot, axis=0, keepdims=True)

    @pl.when(i == NBLK - 1)
    def _epilogue():
        M = acc_scr[0, 0] * (1.0 / N)
        counts = cnt_scr[...]
        probs = counts / (jnp.float32(N) + 1e-8)
        entropy = -jnp.sum(probs * jnp.log(probs + 1e-8))
        commit_ref[...] = jnp.full((1, 1), COMMIT_W * M, _F32)
        cb_ref[...] = jnp.full((1, 1), M - ENT_LAMBDA * entropy, _F32)


_tc_call = pl.pallas_call(
    _tc_body,
    grid=(NBLK,),
    in_specs=[
        pl.BlockSpec((BR, D), lambda i: (i, 0)),
        pl.BlockSpec((D, P), lambda i: (0, 0)),
        pl.BlockSpec((D, D), lambda i: (0, 0)),
    ],
    out_specs=(
        pl.BlockSpec((BR, 1), lambda i: (i, 0)),
        pl.BlockSpec((1, 1), lambda i: (0, 0)),
        pl.BlockSpec((1, 1), lambda i: (0, 0)),
    ),
    out_shape=(
        jax.ShapeDtypeStruct((N, 1), jnp.int32),
        jax.ShapeDtypeStruct((1, 1), _F32),
        jax.ShapeDtypeStruct((1, 1), _F32),
    ),
    scratch_shapes=[
        pltpu.VMEM((1, P), _F32),
        pltpu.VMEM((1, 1), _F32),
    ],
)


def _sc_gather(codebook, idx):
    info = plsc.get_sparse_core_info()
    nw = info.num_cores * info.num_subcores
    b_per_w = N // nw
    mesh = plsc.VectorSubcoreMesh(core_axis_name="c", subcore_axis_name="s")

    @functools.partial(
        pl.kernel, mesh=mesh,
        out_type=jax.ShapeDtypeStruct((N, D), _F32),
        scratch_types=[
            pltpu.VMEM((b_per_w,), jnp.int32),
            pltpu.VMEM((b_per_w, D), _F32),
            pltpu.SemaphoreType.DMA,
        ],
    )
    def k(table_hbm, idx_hbm, out_hbm, idx_v, rows_v, sem):
        wid = lax.axis_index("s") * info.num_cores + lax.axis_index("c")
        base = wid * b_per_w
        pltpu.sync_copy(idx_hbm.at[pl.ds(base, b_per_w)], idx_v)
        pltpu.async_copy(table_hbm.at[idx_v], rows_v, sem).wait()
        pltpu.sync_copy(rows_v, out_hbm.at[pl.ds(base, b_per_w)])

    return k(codebook, idx)


def kernel(input_data, codebook):
    mu = jnp.mean(codebook, axis=0, keepdims=True)
    cen = codebook - mu
    cov = cen.T @ cen / (P - 1)
    cov = cov + 0.001 * jnp.eye(D, dtype=_F32)
    inv_cov = jax.lax.stop_gradient(jnp.linalg.inv(cov))

    idx2d, commit, cb = _tc_call(input_data, codebook.T, inv_cov)
    idx = idx2d.reshape(N)
    quantized = _sc_gather(codebook, idx)
    return (quantized, commit[0, 0], cb[0, 0], idx)


# speedup vs baseline: 1.1971x; 1.0315x over previous
import functools

import jax
import jax.numpy as jnp
from jax import lax
from jax.experimental import pallas as pl
from jax.experimental.pallas import tpu as pltpu
from jax.experimental.pallas import tpu_sc as plsc

P = 512
D = 256
N = 1024
COMMIT_W = 0.1
ENT_LAMBDA = 200000.0
BR = 64
NBLK = N // BR

_F32 = jnp.float32


def _seq_reduce_t(prodT):
    acc = prodT[0:8, :]
    for k in range(1, D // 8):
        acc = acc + prodT[8 * k:8 * k + 8, :]
    s = acc[0:4, :] + acc[4:8, :]
    s = s[0:2, :] + s[2:4, :]
    return s[0:1, :] + s[1:2, :]


def _tc_body(x_ref, ct_ref, a_ref, idx_ref, commit_ref, cb_ref,
             cnt_scr, acc_scr):
    i = pl.program_id(0)
    ccT = ct_ref[...]
    aa = a_ref[...]

    @pl.when(i == 0)
    def _prologue():
        cnt_scr[...] = jnp.zeros((1, P), _F32)
        acc_scr[...] = jnp.zeros((1, 1), _F32)

    rows = []
    for r in range(BR):
        col = x_ref[r:r + 1, :]
        diffT = jnp.broadcast_to(col.reshape(D, 1), (D, P)) - ccT
        leftT = lax.dot_general(aa, diffT, (((0,), (0,)), ((), ())),
                                preferred_element_type=_F32)
        rows.append(_seq_reduce_t(leftT * diffT))
    scores = jnp.concatenate(rows, axis=0)

    smin = jnp.min(scores, axis=1, keepdims=True)
    pio = lax.broadcasted_iota(jnp.int32, (BR, P), 1)
    idx = jnp.min(jnp.where(scores == smin, pio, P), axis=1, keepdims=True)
    idx_ref[...] = idx

    acc_scr[...] += jnp.full((1, 1), jnp.sum(smin), _F32)
    onehot = (idx == pio).astype(_F32)
    cnt_scr[...] += jnp.sum(onehot, axis=0, keepdims=True)

    @pl.when(i == NBLK - 1)
    def _epilogue():
        M = acc_scr[0, 0] * (1.0 / N)
        counts = cnt_scr[...]
        probs = counts / (jnp.float32(N) + 1e-8)
        entropy = -jnp.sum(probs * jnp.log(probs + 1e-8))
        commit_ref[...] = jnp.full((1, 1), COMMIT_W * M, _F32)
        cb_ref[...] = jnp.full((1, 1), M - ENT_LAMBDA * entropy, _F32)


_tc_call = pl.pallas_call(
    _tc_body,
    grid=(NBLK,),
    in_specs=[
        pl.BlockSpec((BR, D), lambda i: (i, 0)),
        pl.BlockSpec((D, P), lambda i: (0, 0)),
        pl.BlockSpec((D, D), lambda i: (0, 0)),
    ],
    out_specs=(
        pl.BlockSpec((BR, 1), lambda i: (i, 0)),
        pl.BlockSpec((1, 1), lambda i: (0, 0)),
        pl.BlockSpec((1, 1), lambda i: (0, 0)),
    ),
    out_shape=(
        jax.ShapeDtypeStruct((N, 1), jnp.int32),
        jax.ShapeDtypeStruct((1, 1), _F32),
        jax.ShapeDtypeStruct((1, 1), _F32),
    ),
    scratch_shapes=[
        pltpu.VMEM((1, P), _F32),
        pltpu.VMEM((1, 1), _F32),
    ],
)


def _sc_gather(codebook, idx):
    info = plsc.get_sparse_core_info()
    nw = info.num_cores * info.num_subcores
    b_per_w = N // nw
    mesh = plsc.VectorSubcoreMesh(core_axis_name="c", subcore_axis_name="s")

    @functools.partial(
        pl.kernel, mesh=mesh,
        out_type=jax.ShapeDtypeStruct((N, D), _F32),
        scratch_types=[
            pltpu.VMEM((b_per_w,), jnp.int32),
            pltpu.VMEM((b_per_w, D), _F32),
            pltpu.SemaphoreType.DMA,
        ],
    )
    def k(table_hbm, idx_hbm, out_hbm, idx_v, rows_v, sem):
        wid = lax.axis_index("s") * info.num_cores + lax.axis_index("c")
        base = wid * b_per_w
        pltpu.sync_copy(idx_hbm.at[pl.ds(base, b_per_w)], idx_v)
        pltpu.async_copy(table_hbm.at[idx_v], rows_v, sem).wait()
        pltpu.sync_copy(rows_v, out_hbm.at[pl.ds(base, b_per_w)])

    return k(codebook, idx)


def kernel(input_data, codebook):
    mu = jnp.mean(codebook, axis=0, keepdims=True)
    cen = codebook - mu
    cov = cen.T @ cen / (P - 1)
    cov = cov + 0.001 * jnp.eye(D, dtype=_F32)
    inv_cov = jax.lax.stop_gradient(jnp.linalg.inv(cov))

    idx2d, commit, cb = _tc_call(input_data, codebook.T, inv_cov)
    idx = idx2d.reshape(N)
    quantized = _sc_gather(codebook, idx)
    return (quantized, commit[0, 0], cb[0, 0], idx)


# speedup vs baseline: 1.2146x; 1.0146x over previous
import functools

import jax
import jax.numpy as jnp
from jax import lax
from jax.experimental import pallas as pl
from jax.experimental.pallas import tpu as pltpu
from jax.experimental.pallas import tpu_sc as plsc

P = 512
D = 256
N = 1024
COMMIT_W = 0.1
ENT_LAMBDA = 200000.0
BR = 128
NBLK = N // BR

_F32 = jnp.float32


def _seq_reduce_t(prodT):
    acc = prodT[0:8, :]
    for k in range(1, D // 8):
        acc = acc + prodT[8 * k:8 * k + 8, :]
    s = acc[0:4, :] + acc[4:8, :]
    s = s[0:2, :] + s[2:4, :]
    return s[0:1, :] + s[1:2, :]


def _tc_body(x_ref, ct_ref, a_ref, idx_ref, commit_ref, cb_ref,
             cnt_scr, acc_scr):
    i = pl.program_id(0)
    ccT = ct_ref[...]
    aa = a_ref[...]

    @pl.when(i == 0)
    def _prologue():
        cnt_scr[...] = jnp.zeros((1, P), _F32)
        acc_scr[...] = jnp.zeros((1, 1), _F32)

    rows = []
    for r in range(BR):
        col = x_ref[r:r + 1, :]
        diffT = jnp.broadcast_to(col.reshape(D, 1), (D, P)) - ccT
        leftT = lax.dot_general(aa, diffT, (((0,), (0,)), ((), ())),
                                preferred_element_type=_F32)
        rows.append(_seq_reduce_t(leftT * diffT))
    scores = jnp.concatenate(rows, axis=0)

    smin = jnp.min(scores, axis=1, keepdims=True)
    pio = lax.broadcasted_iota(jnp.int32, (BR, P), 1)
    idx = jnp.min(jnp.where(scores == smin, pio, P), axis=1, keepdims=True)
    idx_ref[...] = idx

    acc_scr[...] += jnp.full((1, 1), jnp.sum(smin), _F32)
    onehot = (idx == pio).astype(_F32)
    cnt_scr[...] += jnp.sum(onehot, axis=0, keepdims=True)

    @pl.when(i == NBLK - 1)
    def _epilogue():
        M = acc_scr[0, 0] * (1.0 / N)
        counts = cnt_scr[...]
        probs = counts / (jnp.float32(N) + 1e-8)
        entropy = -jnp.sum(probs * jnp.log(probs + 1e-8))
        commit_ref[...] = jnp.full((1, 1), COMMIT_W * M, _F32)
        cb_ref[...] = jnp.full((1, 1), M - ENT_LAMBDA * entropy, _F32)


_tc_call = pl.pallas_call(
    _tc_body,
    grid=(NBLK,),
    in_specs=[
        pl.BlockSpec((BR, D), lambda i: (i, 0)),
        pl.BlockSpec((D, P), lambda i: (0, 0)),
        pl.BlockSpec((D, D), lambda i: (0, 0)),
    ],
    out_specs=(
        pl.BlockSpec((BR, 1), lambda i: (i, 0)),
        pl.BlockSpec((1, 1), lambda i: (0, 0)),
        pl.BlockSpec((1, 1), lambda i: (0, 0)),
    ),
    out_shape=(
        jax.ShapeDtypeStruct((N, 1), jnp.int32),
        jax.ShapeDtypeStruct((1, 1), _F32),
        jax.ShapeDtypeStruct((1, 1), _F32),
    ),
    scratch_shapes=[
        pltpu.VMEM((1, P), _F32),
        pltpu.VMEM((1, 1), _F32),
    ],
)


def _sc_gather(codebook, idx):
    info = plsc.get_sparse_core_info()
    nw = info.num_cores * info.num_subcores
    b_per_w = N // nw
    mesh = plsc.VectorSubcoreMesh(core_axis_name="c", subcore_axis_name="s")

    @functools.partial(
        pl.kernel, mesh=mesh,
        out_type=jax.ShapeDtypeStruct((N, D), _F32),
        scratch_types=[
            pltpu.VMEM((b_per_w,), jnp.int32),
            pltpu.VMEM((b_per_w, D), _F32),
            pltpu.SemaphoreType.DMA,
        ],
    )
    def k(table_hbm, idx_hbm, out_hbm, idx_v, rows_v, sem):
        wid = lax.axis_index("s") * info.num_cores + lax.axis_index("c")
        base = wid * b_per_w
        pltpu.sync_copy(idx_hbm.at[pl.ds(base, b_per_w)], idx_v)
        pltpu.async_copy(table_hbm.at[idx_v], rows_v, sem).wait()
        pltpu.sync_copy(rows_v, out_hbm.at[pl.ds(base, b_per_w)])

    return k(codebook, idx)


def kernel(input_data, codebook):
    mu = jnp.mean(codebook, axis=0, keepdims=True)
    cen = codebook - mu
    cov = cen.T @ cen / (P - 1)
    cov = cov + 0.001 * jnp.eye(D, dtype=_F32)
    inv_cov = jax.lax.stop_gradient(jnp.linalg.inv(cov))

    idx2d, commit, cb = _tc_call(input_data, codebook.T, inv_cov)
    idx = idx2d.reshape(N)
    quantized = _sc_gather(codebook, idx)
    return (quantized, commit[0, 0], cb[0, 0], idx)
